# baseline (speedup 1.0000x reference)
import functools

import jax
import jax.numpy as jnp
from jax import lax
from jax.experimental import pallas as pl
from jax.experimental.pallas import tpu as pltpu
from jax.experimental.pallas import tpu_sc as plsc

V = 100000
D = 64
R = 16
B = 16384
L = 20

_NC = 2
_NS = 16
_NW = _NC * _NS

_BW = B // _NW
_RPD = 128
_CPL = _BW // _RPD

_TBL_BLK = 8192
_LANES = 128


def _build_table_body(orig_t_ref, aw1_t_ref, m_ref, out_ref):
    cat = jnp.concatenate([orig_t_ref[...], aw1_t_ref[...]], axis=0)
    out_ref[:, 0:D] = lax.dot_general(
        cat, m_ref[...],
        dimension_numbers=(((0,), (0,)), ((), ())),
        preferred_element_type=jnp.float32,
    )


def _build_table(orig_t, aw1_t, m):
    return pl.pallas_call(
        _build_table_body,
        grid=(pl.cdiv(V, _TBL_BLK),),
        in_specs=[
            pl.BlockSpec((D, _TBL_BLK), lambda i: (0, i)),
            pl.BlockSpec((R, _TBL_BLK), lambda i: (0, i)),
            pl.BlockSpec((D + R, D), lambda i: (0, 0)),
        ],
        out_specs=pl.BlockSpec((_TBL_BLK, _LANES), lambda i: (i, 0)),
        out_shape=jax.ShapeDtypeStruct((V, _LANES), jnp.float32),
    )(orig_t, aw1_t, m)


@functools.partial(
    pl.kernel,
    mesh=plsc.VectorSubcoreMesh(core_axis_name="c", subcore_axis_name="s"),
    out_type=jax.ShapeDtypeStruct((L, D, B), jnp.float32),
    scratch_types=[
        pltpu.VMEM((L, _BW), jnp.int32),
        pltpu.VMEM((2, _RPD, _LANES), jnp.float32),
        pltpu.VMEM((2, D, _RPD), jnp.float32),
        pltpu.SemaphoreType.DMA,
        pltpu.SemaphoreType.DMA,
        pltpu.SemaphoreType.DMA,
        pltpu.SemaphoreType.DMA,
    ],
    compiler_params=pltpu.CompilerParams(needs_layout_passes=False),
)
def _gather(table_hbm, xt_hbm, out_hbm, idx_v, rows_v, comp_v,
            sg0, sg1, so0, so1):
    wid = lax.axis_index("s") * _NC + lax.axis_index("c")
    b0 = wid * _BW
    pltpu.sync_copy(xt_hbm.at[:, pl.ds(b0, _BW)], idx_v)
    sg = (sg0, sg1)
    so = (so0, so1)
    n_ch = L * _CPL

    def gather_cp(j, b, sem):
        l = j // _CPL
        c = j % _CPL
        return pltpu.make_async_copy(
            table_hbm.at[idx_v.at[l, pl.ds(c * _RPD, _RPD)]],
            rows_v.at[b], sem,
        )

    def out_cp(j, b, sem):
        l = j // _CPL
        c = j % _CPL
        return pltpu.make_async_copy(
            comp_v.at[b],
            out_hbm.at[l, :, pl.ds(b0 + c * _RPD, _RPD)],
            sem,
        )

    lane = lax.iota(jnp.int32, 16)
    row_g = [lane + g * 16 for g in range(_RPD // 16)]

    gather_cp(0, 0, sg[0]).start()

    def outer(j0, carry):
        for b in range(2):
            j = j0 + b
            gather_cp(j, b, sg[b]).wait()

            @pl.when(j + 1 < n_ch)
            def _():
                gather_cp(j + 1, 1 - b, sg[1 - b]).start()

            @pl.when(j >= 2)
            def _():
                out_cp(j - 2, b, so[b]).wait()

            def drow(d, c):
                dcol = lax.broadcast_in_dim(d, (16,), ())
                for g in range(_RPD // 16):
                    v16 = plsc.load_gather(rows_v.at[b], [row_g[g], dcol])
                    comp_v[b, d, pl.ds(g * 16, 16)] = v16
                return c

            lax.fori_loop(0, D, drow, 0)
            out_cp(j, b, so[b]).start()
        return carry

    lax.fori_loop(0, n_ch // 2, lambda i, c: outer(i * 2, c), 0)
    out_cp(n_ch - 2, 0, so[0]).wait()
    out_cp(n_ch - 1, 1, so[1]).wait()


def kernel(x, orig_weight, aw1, aw2):
    m = jnp.concatenate([jnp.eye(D, dtype=jnp.float32), aw2], axis=0)
    table = _build_table(orig_weight.T, aw1.T, m)
    out_phys = _gather(table, x.T.astype(jnp.int32))
    return jnp.transpose(out_phys, (2, 0, 1))


# speedup vs baseline: 1.4120x; 1.4120x over previous
import functools

import jax
import jax.numpy as jnp
from jax import lax
from jax.experimental import pallas as pl
from jax.experimental.pallas import tpu as pltpu
from jax.experimental.pallas import tpu_sc as plsc

V = 100000
D = 64
R = 16
B = 16384
L = 20

_NC = 2
_NS = 16
_NW = _NC * _NS

_BW = B // _NW
_RPD = 128
_CPL = _BW // _RPD

_TBL_BLK = 8192
_LANES = 128


def _build_table_body(orig_t_ref, aw1_t_ref, m_ref, out_ref):
    cat = jnp.concatenate([orig_t_ref[...], aw1_t_ref[...]], axis=0)
    out_ref[:, 0:D] = lax.dot_general(
        cat, m_ref[...],
        dimension_numbers=(((0,), (0,)), ((), ())),
        preferred_element_type=jnp.float32,
    )


def _build_table(orig_t, aw1_t, m):
    return pl.pallas_call(
        _build_table_body,
        grid=(pl.cdiv(V, _TBL_BLK),),
        in_specs=[
            pl.BlockSpec((D, _TBL_BLK), lambda i: (0, i)),
            pl.BlockSpec((R, _TBL_BLK), lambda i: (0, i)),
            pl.BlockSpec((D + R, D), lambda i: (0, 0)),
        ],
        out_specs=pl.BlockSpec((_TBL_BLK, _LANES), lambda i: (i, 0)),
        out_shape=jax.ShapeDtypeStruct((V, _LANES), jnp.float32),
    )(orig_t, aw1_t, m)


@functools.partial(
    pl.kernel,
    mesh=plsc.VectorSubcoreMesh(core_axis_name="c", subcore_axis_name="s"),
    out_type=jax.ShapeDtypeStruct((L, B // 2, _LANES), jnp.float32),
    scratch_types=[
        pltpu.VMEM((L, _BW), jnp.int32),
        pltpu.VMEM((2, _RPD, _LANES), jnp.float32),
        pltpu.VMEM((2, _RPD // 2, _LANES), jnp.float32),
        pltpu.SemaphoreType.DMA,
        pltpu.SemaphoreType.DMA,
        pltpu.SemaphoreType.DMA,
        pltpu.SemaphoreType.DMA,
    ],
)
def _gather(table_hbm, xt_hbm, out_hbm, idx_v, rows_v, comp_v,
            sg0, sg1, so0, so1):
    wid = lax.axis_index("s") * _NC + lax.axis_index("c")
    b0 = wid * _BW
    pltpu.sync_copy(xt_hbm.at[:, pl.ds(b0, _BW)], idx_v)
    sg = (sg0, sg1)
    so = (so0, so1)
    n_ch = L * _CPL

    def gather_cp(j, b, sem):
        l = j // _CPL
        c = j % _CPL
        return pltpu.make_async_copy(
            table_hbm.at[idx_v.at[l, pl.ds(c * _RPD, _RPD)]],
            rows_v.at[b], sem,
        )

    def out_cp(j, b, sem):
        l = j // _CPL
        c = j % _CPL
        return pltpu.make_async_copy(
            comp_v.at[b],
            out_hbm.at[l, pl.ds(pl.multiple_of((b0 + c * _RPD) // 2, 64),
                                _RPD // 2)],
            sem,
        )

    gather_cp(0, 0, sg[0]).start()

    def outer(j0, carry):
        for b in range(2):
            j = j0 + b
            gather_cp(j, b, sg[b]).wait()

            @pl.when(j + 1 < n_ch)
            def _():
                gather_cp(j + 1, 1 - b, sg[1 - b]).start()

            @pl.when(j >= 2)
            def _():
                out_cp(j - 2, b, so[b]).wait()

            def tok(t, c):
                half = (t & 1) * D
                for q in range(D // 16):
                    comp_v[b, t >> 1, pl.ds(half + q * 16, 16)] = rows_v[
                        b, t, pl.ds(q * 16, 16)
                    ]
                return c

            lax.fori_loop(0, _RPD, tok, 0)
            out_cp(j, b, so[b]).start()
        return carry

    lax.fori_loop(0, n_ch // 2, lambda i, c: outer(i * 2, c), 0)
    out_cp(n_ch - 2, 0, so[0]).wait()
    out_cp(n_ch - 1, 1, so[1]).wait()


def kernel(x, orig_weight, aw1, aw2):
    m = jnp.concatenate([jnp.eye(D, dtype=jnp.float32), aw2], axis=0)
    table = _build_table(orig_weight.T, aw1.T, m)
    out_lb = _gather(table, x.T.astype(jnp.int32))
    return jnp.transpose(out_lb.reshape(L, B, D), (1, 0, 2))


# speedup vs baseline: 2.1611x; 1.5305x over previous
import functools

import jax
import jax.numpy as jnp
from jax import lax
from jax.experimental import pallas as pl
from jax.experimental.pallas import tpu as pltpu
from jax.experimental.pallas import tpu_sc as plsc

V = 100000
D = 64
R = 16
B = 16384
L = 20

_NC = 2
_NS = 16
_NW = _NC * _NS

_BW = B // _NW
_RPD = 128
_CPL = _BW // _RPD

_TBL_BLK = 8192
_LANES = 128


def _build_table_body(orig_t_ref, aw1_t_ref, m_ref, out_ref):
    cat = jnp.concatenate([orig_t_ref[...], aw1_t_ref[...]], axis=0)
    out_ref[:, 0:D] = lax.dot_general(
        cat, m_ref[...],
        dimension_numbers=(((0,), (0,)), ((), ())),
        preferred_element_type=jnp.float32,
    )


def _build_table(orig_t, aw1_t, m):
    return pl.pallas_call(
        _build_table_body,
        grid=(pl.cdiv(V, _TBL_BLK),),
        in_specs=[
            pl.BlockSpec((D, _TBL_BLK), lambda i: (0, i)),
            pl.BlockSpec((R, _TBL_BLK), lambda i: (0, i)),
            pl.BlockSpec((D + R, D), lambda i: (0, 0)),
        ],
        out_specs=pl.BlockSpec((_TBL_BLK, _LANES), lambda i: (i, 0)),
        out_shape=jax.ShapeDtypeStruct((V, _LANES), jnp.float32),
    )(orig_t, aw1_t, m)


@functools.partial(
    pl.kernel,
    mesh=plsc.VectorSubcoreMesh(core_axis_name="c", subcore_axis_name="s"),
    out_type=jax.ShapeDtypeStruct((L, B, D), jnp.float32),
    scratch_types=[
        pltpu.VMEM((L, _BW), jnp.int32),
        pltpu.VMEM((2, _RPD, _LANES), jnp.float32),
        pltpu.VMEM((2, _RPD, D), jnp.float32),
        pltpu.SemaphoreType.DMA,
        pltpu.SemaphoreType.DMA,
        pltpu.SemaphoreType.DMA,
        pltpu.SemaphoreType.DMA,
    ],
)
def _gather(table_hbm, xt_hbm, out_hbm, idx_v, rows_v, comp_v,
            sg0, sg1, so0, so1):
    wid = lax.axis_index("s") * _NC + lax.axis_index("c")
    b0 = wid * _BW
    pltpu.sync_copy(xt_hbm.at[:, pl.ds(b0, _BW)], idx_v)
    sg = (sg0, sg1)
    so = (so0, so1)
    n_ch = L * _CPL

    def gather_cp(j, b, sem):
        l = j // _CPL
        c = j % _CPL
        return pltpu.make_async_copy(
            table_hbm.at[idx_v.at[l, pl.ds(c * _RPD, _RPD)]],
            rows_v.at[b], sem,
        )

    def out_cp(j, b, sem):
        l = j // _CPL
        c = j % _CPL
        return pltpu.make_async_copy(
            comp_v.at[b],
            out_hbm.at[l, pl.ds(pl.multiple_of(b0 + c * _RPD, _RPD), _RPD), :],
            sem,
        )

    gather_cp(0, 0, sg[0]).start()

    def outer(j0, carry):
        for b in range(2):
            j = j0 + b
            gather_cp(j, b, sg[b]).wait()

            @pl.when(j + 1 < n_ch)
            def _():
                gather_cp(j + 1, 1 - b, sg[1 - b]).start()

            @pl.when(j >= 2)
            def _():
                out_cp(j - 2, b, so[b]).wait()

            def tok(t, c):
                for q in range(D // 16):
                    comp_v[b, t, pl.ds(q * 16, 16)] = rows_v[
                        b, t, pl.ds(q * 16, 16)
                    ]
                return c

            lax.fori_loop(0, _RPD, tok, 0)
            out_cp(j, b, so[b]).start()
        return carry

    lax.fori_loop(0, n_ch // 2, lambda i, c: outer(i * 2, c), 0)
    out_cp(n_ch - 2, 0, so[0]).wait()
    out_cp(n_ch - 1, 1, so[1]).wait()


def kernel(x, orig_weight, aw1, aw2):
    m = jnp.concatenate([jnp.eye(D, dtype=jnp.float32), aw2], axis=0)
    table = _build_table(orig_weight.T, aw1.T, m)
    out_lb = _gather(table, x.T.astype(jnp.int32))
    return jnp.transpose(out_lb, (1, 0, 2))
